# baseline (speedup 1.0000x reference)
import functools

import numpy as np
import jax
import jax.numpy as jnp
from jax.experimental import pallas as pl
from jax.experimental.pallas import tpu as pltpu
from jax.experimental.pallas import tpu_sc as plsc

_B = 64
_N = 1000000

_W = 16384
_C = 720896
_NB = _C // _W
_SW = 512
_NSUB = _W // _SW

_TB = 16384
_NCH = 18
_P = _NCH * _TB
_TSUB = _TB // _SW

_K0 = np.uint32(0)
_K1 = np.uint32(42)
_K2 = np.uint32(np.uint32(0x1BD11BDA) ^ _K0 ^ _K1)
_KS = (_K0, _K1, _K2)
_ROT = ((13, 15, 26, 6), (17, 29, 16, 24))
_TINY = np.float32(np.finfo(np.float32).tiny)


def _threefry_bits(x1):
    x1 = x1 + _K1
    x0 = x1
    x1 = ((x1 << np.uint32(13)) | (x1 >> np.uint32(19))) ^ x0
    first = True
    for i in range(5):
        for r in _ROT[i % 2]:
            if first:
                first = False
                continue
            x0 = x0 + x1
            x1 = (x1 << np.uint32(r)) | (x1 >> np.uint32(32 - r))
            x1 = x1 ^ x0
        x0 = x0 + _KS[(i + 1) % 3]
        x1 = x1 + np.uint32(_KS[(i + 2) % 3] + np.uint32(i + 1))
    return x0 ^ x1


def _gumbel_from_bits(bits):
    fl = jax.lax.bitcast_convert_type(
        (bits >> np.uint32(9)) | np.uint32(0x3F800000), jnp.float32
    ) - np.float32(1.0)
    return -jnp.log(-jnp.log(fl + _TINY))


def _main_body(logits_ref, outv_ref, outi_ref, bestv_ref, besti_ref):
    j = pl.program_id(0)

    @pl.when(j == 0)
    def _init():
        bestv_ref[...] = jnp.full((_B, 1), -jnp.inf, jnp.float32)
        besti_ref[...] = jnp.zeros((_B, 1), jnp.int32)

    rowbase = jax.lax.broadcasted_iota(jnp.uint32, (_B, _SW), 0) * jnp.uint32(_N)
    lane = jax.lax.broadcasted_iota(jnp.uint32, (_B, _SW), 1)
    jbase = jnp.uint32(_W) * j.astype(jnp.uint32)

    sv = jnp.full((_B, _SW), -jnp.inf, jnp.float32)
    si = jnp.zeros((_B, _SW), jnp.uint32)
    for k in range(_NSUB):
        col = lane + (jbase + jnp.uint32(k * _SW))
        flat = rowbase + col
        g = _gumbel_from_bits(_threefry_bits(flat))
        s = logits_ref[:, k * _SW:(k + 1) * _SW] + g
        upd = s > sv
        sv = jnp.where(upd, s, sv)
        si = jnp.where(upd, flat, si)

    m = jnp.max(sv, axis=1, keepdims=True)
    idx = jnp.min(
        jnp.where(sv == m, si.astype(jnp.int32), jnp.int32(2**31 - 1)),
        axis=1,
        keepdims=True,
    )
    upd = m > bestv_ref[...]
    besti_ref[...] = jnp.where(upd, idx, besti_ref[...])
    bestv_ref[...] = jnp.where(upd, m, bestv_ref[...])

    @pl.when(j == _NB - 1)
    def _emit():
        rown = jax.lax.broadcasted_iota(jnp.int32, (_B, 1), 0) * jnp.int32(_N)
        outv_ref[...] = bestv_ref[...]
        outi_ref[...] = besti_ref[...] - rown


def _main_call(logits):
    return pl.pallas_call(
        _main_body,
        grid=(_NB,),
        in_specs=[pl.BlockSpec((_B, _W), lambda j: (0, j))],
        out_specs=[
            pl.BlockSpec((_B, 1), lambda j: (0, 0)),
            pl.BlockSpec((_B, 1), lambda j: (0, 0)),
        ],
        out_shape=[
            jax.ShapeDtypeStruct((_B, 1), jnp.float32),
            jax.ShapeDtypeStruct((_B, 1), jnp.int32),
        ],
        scratch_shapes=[
            pltpu.VMEM((_B, 1), jnp.float32),
            pltpu.VMEM((_B, 1), jnp.int32),
        ],
    )(logits)


@functools.lru_cache(maxsize=1)
def _sc_bits_fn():
    mesh = plsc.VectorSubcoreMesh(
        core_axis_name="c", subcore_axis_name="s", num_cores=2, num_subcores=16
    )

    @functools.partial(
        pl.kernel,
        out_type=jax.ShapeDtypeStruct((_NCH, _B, _TB), jnp.uint32),
        mesh=mesh,
        scratch_types=[pltpu.VMEM((_TB,), jnp.uint32)],
    )
    def _sc_bits(out_hbm, buf):
        wid = jax.lax.axis_index("s") * 2 + jax.lax.axis_index("c")
        lane16 = jax.lax.iota(jnp.uint32, 16)
        for r in range(2):
            row = wid * 2 + r
            base = row.astype(jnp.uint32) * jnp.uint32(_N) + jnp.uint32(_C)

            def _chunk(i, _, base=base, row=row):
                cbase = base + jnp.uint32(_TB) * i.astype(jnp.uint32)

                def _vec(v, _, cbase=cbase):
                    x1 = lane16 + (cbase + jnp.uint32(16) * v.astype(jnp.uint32))
                    buf[pl.ds(v * 16, 16)] = _threefry_bits(x1)
                    return ()

                jax.lax.fori_loop(0, _TB // 16, _vec, (), unroll=8)
                pltpu.sync_copy(buf, out_hbm.at[i, row])
                return ()

            jax.lax.fori_loop(0, _NCH, _chunk, ())

    return _sc_bits


def _tail_body(bits_ref, logits_ref, outv_ref, outi_ref, bestv_ref, besti_ref):
    j = pl.program_id(0)

    @pl.when(j == 0)
    def _init():
        bestv_ref[...] = jnp.full((_B, 1), -jnp.inf, jnp.float32)
        besti_ref[...] = jnp.zeros((_B, 1), jnp.int32)

    rowbase = jax.lax.broadcasted_iota(jnp.uint32, (_B, _SW), 0) * jnp.uint32(_N)
    lane = jax.lax.broadcasted_iota(jnp.uint32, (_B, _SW), 1)
    jbase = jnp.uint32(_C) + jnp.uint32(_TB) * j.astype(jnp.uint32)

    sv = jnp.full((_B, _SW), -jnp.inf, jnp.float32)
    si = jnp.zeros((_B, _SW), jnp.uint32)
    for k in range(_TSUB):
        col = lane + (jbase + jnp.uint32(k * _SW))
        flat = rowbase + col
        g = _gumbel_from_bits(bits_ref[0, :, k * _SW:(k + 1) * _SW])
        s = logits_ref[:, k * _SW:(k + 1) * _SW] + g
        s = jnp.where(col < jnp.uint32(_N), s, -jnp.inf)
        upd = s > sv
        sv = jnp.where(upd, s, sv)
        si = jnp.where(upd, flat, si)

    m = jnp.max(sv, axis=1, keepdims=True)
    idx = jnp.min(
        jnp.where(sv == m, si.astype(jnp.int32), jnp.int32(2**31 - 1)),
        axis=1,
        keepdims=True,
    )
    upd = m > bestv_ref[...]
    besti_ref[...] = jnp.where(upd, idx, besti_ref[...])
    bestv_ref[...] = jnp.where(upd, m, bestv_ref[...])

    @pl.when(j == _NCH - 1)
    def _emit():
        rown = jax.lax.broadcasted_iota(jnp.int32, (_B, 1), 0) * jnp.int32(_N)
        outv_ref[...] = bestv_ref[...]
        outi_ref[...] = besti_ref[...] - rown


def _tail_call(bits, logits):
    return pl.pallas_call(
        _tail_body,
        grid=(_NCH,),
        in_specs=[
            pl.BlockSpec((1, _B, _TB), lambda j: (j, 0, 0)),
            pl.BlockSpec((_B, _TB), lambda j: (0, j + _C // _TB)),
        ],
        out_specs=[
            pl.BlockSpec((_B, 1), lambda j: (0, 0)),
            pl.BlockSpec((_B, 1), lambda j: (0, 0)),
        ],
        out_shape=[
            jax.ShapeDtypeStruct((_B, 1), jnp.float32),
            jax.ShapeDtypeStruct((_B, 1), jnp.int32),
        ],
        scratch_shapes=[
            pltpu.VMEM((_B, 1), jnp.float32),
            pltpu.VMEM((_B, 1), jnp.int32),
        ],
    )(bits, logits)


def kernel(logits):
    bits = _sc_bits_fn()()
    bv1, bi1 = _main_call(logits)
    bv2, bi2 = _tail_call(bits, logits)
    out = jnp.where(bv2 > bv1, bi2, bi1)
    return out.reshape(_B)


# speedup vs baseline: 1.0142x; 1.0142x over previous
import functools

import numpy as np
import jax
import jax.numpy as jnp
from jax.experimental import pallas as pl
from jax.experimental.pallas import tpu as pltpu
from jax.experimental.pallas import tpu_sc as plsc

_B = 64
_N = 1000000

_W = 16384
_C = 720896
_NB = _C // _W
_SW = 256
_NSUB = _W // _SW

_TB = 8192
_NCH = 35
_P = _NCH * _TB
_TSUB = _TB // _SW

_K0 = np.uint32(0)
_K1 = np.uint32(42)
_K2 = np.uint32(np.uint32(0x1BD11BDA) ^ _K0 ^ _K1)
_KS = (_K0, _K1, _K2)
_ROT = ((13, 15, 26, 6), (17, 29, 16, 24))
_TINY = np.float32(np.finfo(np.float32).tiny)


def _threefry_bits(x1):
    x1 = x1 + _K1
    x0 = x1
    x1 = ((x1 << np.uint32(13)) | (x1 >> np.uint32(19))) ^ x0
    first = True
    for i in range(5):
        for r in _ROT[i % 2]:
            if first:
                first = False
                continue
            x0 = x0 + x1
            x1 = (x1 << np.uint32(r)) | (x1 >> np.uint32(32 - r))
            x1 = x1 ^ x0
        x0 = x0 + _KS[(i + 1) % 3]
        x1 = x1 + np.uint32(_KS[(i + 2) % 3] + np.uint32(i + 1))
    return x0 ^ x1


def _gumbel_from_bits(bits):
    fl = jax.lax.bitcast_convert_type(
        (bits >> np.uint32(9)) | np.uint32(0x3F800000), jnp.float32
    ) - np.float32(1.0)
    return -jnp.log(-jnp.log(fl + _TINY))


def _main_body(logits_ref, outv_ref, outi_ref, bestv_ref, besti_ref):
    j = pl.program_id(0)

    @pl.when(j == 0)
    def _init():
        bestv_ref[...] = jnp.full((_B, 1), -jnp.inf, jnp.float32)
        besti_ref[...] = jnp.zeros((_B, 1), jnp.int32)

    rowbase = jax.lax.broadcasted_iota(jnp.uint32, (_B, _SW), 0) * jnp.uint32(_N)
    lane = jax.lax.broadcasted_iota(jnp.uint32, (_B, _SW), 1)
    jbase = jnp.uint32(_W) * j.astype(jnp.uint32)

    sv = jnp.full((_B, _SW), -jnp.inf, jnp.float32)
    si = jnp.zeros((_B, _SW), jnp.uint32)
    for k in range(_NSUB):
        col = lane + (jbase + jnp.uint32(k * _SW))
        flat = rowbase + col
        g = _gumbel_from_bits(_threefry_bits(flat))
        s = logits_ref[:, k * _SW:(k + 1) * _SW] + g
        upd = s > sv
        sv = jnp.where(upd, s, sv)
        si = jnp.where(upd, flat, si)

    m = jnp.max(sv, axis=1, keepdims=True)
    idx = jnp.min(
        jnp.where(sv == m, si.astype(jnp.int32), jnp.int32(2**31 - 1)),
        axis=1,
        keepdims=True,
    )
    upd = m > bestv_ref[...]
    besti_ref[...] = jnp.where(upd, idx, besti_ref[...])
    bestv_ref[...] = jnp.where(upd, m, bestv_ref[...])

    @pl.when(j == _NB - 1)
    def _emit():
        rown = jax.lax.broadcasted_iota(jnp.int32, (_B, 1), 0) * jnp.int32(_N)
        outv_ref[...] = bestv_ref[...]
        outi_ref[...] = besti_ref[...] - rown


def _main_call(logits):
    return pl.pallas_call(
        _main_body,
        grid=(_NB,),
        in_specs=[pl.BlockSpec((_B, _W), lambda j: (0, j))],
        out_specs=[
            pl.BlockSpec((_B, 1), lambda j: (0, 0)),
            pl.BlockSpec((_B, 1), lambda j: (0, 0)),
        ],
        out_shape=[
            jax.ShapeDtypeStruct((_B, 1), jnp.float32),
            jax.ShapeDtypeStruct((_B, 1), jnp.int32),
        ],
        scratch_shapes=[
            pltpu.VMEM((_B, 1), jnp.float32),
            pltpu.VMEM((_B, 1), jnp.int32),
        ],
    )(logits)


@functools.lru_cache(maxsize=1)
def _sc_bits_fn():
    mesh = plsc.VectorSubcoreMesh(
        core_axis_name="c", subcore_axis_name="s", num_cores=2, num_subcores=16
    )

    @functools.partial(
        pl.kernel,
        out_type=jax.ShapeDtypeStruct((_NCH, _B, _TB), jnp.uint32),
        mesh=mesh,
        scratch_types=[pltpu.VMEM((_TB,), jnp.uint32)],
    )
    def _sc_bits(out_hbm, buf):
        wid = jax.lax.axis_index("s") * 2 + jax.lax.axis_index("c")
        lane16 = jax.lax.iota(jnp.uint32, 16)
        for r in range(2):
            row = wid * 2 + r
            base = row.astype(jnp.uint32) * jnp.uint32(_N) + jnp.uint32(_C)

            def _chunk(i, _, base=base, row=row):
                cbase = base + jnp.uint32(_TB) * i.astype(jnp.uint32)

                def _vec(v, _, cbase=cbase):
                    x1 = lane16 + (cbase + jnp.uint32(16) * v.astype(jnp.uint32))
                    buf[pl.ds(v * 16, 16)] = _threefry_bits(x1)
                    return ()

                jax.lax.fori_loop(0, _TB // 16, _vec, (), unroll=8)
                pltpu.sync_copy(buf, out_hbm.at[i, row])
                return ()

            jax.lax.fori_loop(0, _NCH, _chunk, ())

    return _sc_bits


def _tail_body(bits_ref, logits_ref, outv_ref, outi_ref, bestv_ref, besti_ref):
    j = pl.program_id(0)

    @pl.when(j == 0)
    def _init():
        bestv_ref[...] = jnp.full((_B, 1), -jnp.inf, jnp.float32)
        besti_ref[...] = jnp.zeros((_B, 1), jnp.int32)

    rowbase = jax.lax.broadcasted_iota(jnp.uint32, (_B, _SW), 0) * jnp.uint32(_N)
    lane = jax.lax.broadcasted_iota(jnp.uint32, (_B, _SW), 1)
    jbase = jnp.uint32(_C) + jnp.uint32(_TB) * j.astype(jnp.uint32)

    sv = jnp.full((_B, _SW), -jnp.inf, jnp.float32)
    si = jnp.zeros((_B, _SW), jnp.uint32)
    for k in range(_TSUB):
        col = lane + (jbase + jnp.uint32(k * _SW))
        flat = rowbase + col
        g = _gumbel_from_bits(bits_ref[0, :, k * _SW:(k + 1) * _SW])
        s = logits_ref[:, k * _SW:(k + 1) * _SW] + g
        s = jnp.where(col < jnp.uint32(_N), s, -jnp.inf)
        upd = s > sv
        sv = jnp.where(upd, s, sv)
        si = jnp.where(upd, flat, si)

    m = jnp.max(sv, axis=1, keepdims=True)
    idx = jnp.min(
        jnp.where(sv == m, si.astype(jnp.int32), jnp.int32(2**31 - 1)),
        axis=1,
        keepdims=True,
    )
    upd = m > bestv_ref[...]
    besti_ref[...] = jnp.where(upd, idx, besti_ref[...])
    bestv_ref[...] = jnp.where(upd, m, bestv_ref[...])

    @pl.when(j == _NCH - 1)
    def _emit():
        rown = jax.lax.broadcasted_iota(jnp.int32, (_B, 1), 0) * jnp.int32(_N)
        outv_ref[...] = bestv_ref[...]
        outi_ref[...] = besti_ref[...] - rown


def _tail_call(bits, logits):
    return pl.pallas_call(
        _tail_body,
        grid=(_NCH,),
        in_specs=[
            pl.BlockSpec((1, _B, _TB), lambda j: (j, 0, 0)),
            pl.BlockSpec((_B, _TB), lambda j: (0, j + _C // _TB)),
        ],
        out_specs=[
            pl.BlockSpec((_B, 1), lambda j: (0, 0)),
            pl.BlockSpec((_B, 1), lambda j: (0, 0)),
        ],
        out_shape=[
            jax.ShapeDtypeStruct((_B, 1), jnp.float32),
            jax.ShapeDtypeStruct((_B, 1), jnp.int32),
        ],
        scratch_shapes=[
            pltpu.VMEM((_B, 1), jnp.float32),
            pltpu.VMEM((_B, 1), jnp.int32),
        ],
    )(bits, logits)


def kernel(logits):
    bits = _sc_bits_fn()()
    bv1, bi1 = _main_call(logits)
    bv2, bi2 = _tail_call(bits, logits)
    out = jnp.where(bv2 > bv1, bi2, bi1)
    return out.reshape(_B)
